# baseline (speedup 1.0000x reference)
import functools

import jax
import jax.numpy as jnp
from jax import lax
from jax.experimental import pallas as pl
from jax.experimental.pallas import tpu as pltpu
from jax.experimental.pallas import tpu_sc as plsc

EMB = 32
INNER = 128

NC = 2
NS = 16
NW = NC * NS
BT_PER_W = 4
BW = BT_PER_W * 128


def _sc_gather(table, xt3, batch, t_dim):
    mesh = plsc.VectorSubcoreMesh(core_axis_name="c", subcore_axis_name="s")

    @functools.partial(
        pl.kernel,
        mesh=mesh,
        out_type=jax.ShapeDtypeStruct((t_dim, batch * EMB // 128, 128),
                                      jnp.float32),
        scratch_types=[pltpu.VMEM((BT_PER_W, 128), jnp.int32) for _ in range(2)]
          + [pltpu.VMEM((BW, EMB), jnp.float32) for _ in range(2)]
          + [pltpu.SemaphoreType.DMA for _ in range(6)],
        compiler_params=pltpu.CompilerParams(use_tc_tiling_on_sc=False),
    )
    def body(tab_hbm, xt_hbm, out_hbm,
             idx0, idx1, rows0, rows1, si0, si1, sg0, sg1, ss0, ss1):
        idx = (idx0, idx1)
        rows = (rows0, rows1)
        sem_i = (si0, si1)
        sem_g = (sg0, sg1)
        sem_s = (ss0, ss1)
        wid = lax.axis_index("s") * NC + lax.axis_index("c")
        bt0 = wid * BT_PER_W
        r0 = wid * 128

        def fire_idx(t, p):
            pltpu.async_copy(xt_hbm.at[t, pl.ds(bt0, BT_PER_W)], idx[p],
                             sem_i[p])

        def wait_idx(p):
            pltpu.make_async_copy(xt_hbm.at[0, pl.ds(0, BT_PER_W)], idx[p],
                                  sem_i[p]).wait()

        def fire_gathers(p):
            for q in range(BT_PER_W):
                pltpu.async_copy(tab_hbm.at[idx[p].at[q]],
                                 rows[p].at[pl.ds(q * 128, 128)], sem_g[p])

        def wait_gathers(p):
            pltpu.make_async_copy(tab_hbm.at[pl.ds(0, BW)], rows[p],
                                  sem_g[p]).wait()

        def fire_store(t, p):
            for q in range(BT_PER_W):
                pltpu.async_copy(rows[p].at[pl.ds(q * 128, 128)],
                                 out_hbm.at[t, pl.ds(r0, 128),
                                            pl.ds(q * EMB, EMB)], sem_s[p])

        def wait_store(p):
            for q in range(BT_PER_W):
                pltpu.make_async_copy(rows[p].at[pl.ds(q * 128, 128)],
                                      out_hbm.at[0, pl.ds(0, 128),
                                                 pl.ds(0, EMB)],
                                      sem_s[p]).wait()

        def t_iter(t, p):
            q = 1 - p

            @pl.when(t + 1 < t_dim)
            def _():
                wait_idx(q)

                @pl.when(t >= 1)
                def _():
                    wait_store(q)

                fire_gathers(q)

            wait_gathers(p)

            @pl.when(t + 2 < t_dim)
            def _():
                fire_idx(t + 2, p)

            fire_store(t, p)

        pltpu.sync_copy(xt_hbm.at[0, pl.ds(bt0, BT_PER_W)], idx[0])
        fire_gathers(0)
        fire_idx(1, 1)

        def pair(m, carry):
            t_iter(2 * m, 0)
            t_iter(2 * m + 1, 1)
            return carry

        lax.fori_loop(0, t_dim // 2, pair, 0)
        wait_store(0)
        wait_store(1)

    return body(table, xt3)


def _finish_body(tmp_ref, w1_ref, w2_ref, pos_ref, b1_ref, b2_ref, out_ref):
    _finish_common(tmp_ref, w1_ref, w2_ref, pos_ref, b1_ref, b2_ref, out_ref)


def _finish_body_alias(tmp_ref, w1_ref, w2_ref, pos_ref, b1_ref, b2_ref,
                       alias_ref, out_ref):
    del alias_ref
    _finish_common(tmp_ref, w1_ref, w2_ref, pos_ref, b1_ref, b2_ref, out_ref)


def _finish_common(tmp_ref, w1_ref, w2_ref, pos_ref, b1_ref, b2_ref, out_ref):
    f32 = jnp.float32
    m = jnp.dot(w1_ref[...], w2_ref[...], preferred_element_type=f32)
    ptc = lax.dot_general(m, pos_ref[0], (((0,), (1,)), ((), ())),
                          preferred_element_type=f32)
    ccol = lax.dot_general(w2_ref[...], b1_ref[...], (((0,), (1,)), ((), ())),
                           preferred_element_type=f32)
    col = ptc + ccol + b2_ref[...]
    v = tmp_ref[0]
    qn = v.shape[0]
    for a in range(4):
        va = v[:, 32 * a:32 * (a + 1)]
        ga = lax.dot_general(m, va, (((0,), (1,)), ((), ())),
                             preferred_element_type=f32)
        out_ref[0, :, a * qn:(a + 1) * qn] = ga + col


def _finish(temp4, W1, b1, W2, b2, pos3, batch, total_t, t_off, alias=None):
    t_half = temp4.shape[0]
    in_specs = [
        pl.BlockSpec((1, batch * EMB // 128, 128), lambda t: (t, 0, 0)),
        pl.BlockSpec((EMB, INNER), lambda t: (0, 0)),
        pl.BlockSpec((INNER, EMB), lambda t: (0, 0)),
        pl.BlockSpec((1, 1, EMB), lambda t: (t, 0, 0)),
        pl.BlockSpec((1, INNER), lambda t: (0, 0)),
        pl.BlockSpec((EMB, 1), lambda t: (0, 0)),
    ]
    args = [temp4, W1, W2, pos3, b1.reshape(1, INNER), b2.reshape(EMB, 1)]
    kwargs = {}
    body = _finish_body
    if alias is not None:
        in_specs.append(pl.BlockSpec(memory_space=pl.ANY))
        args.append(alias)
        kwargs["input_output_aliases"] = {6: 0}
        body = _finish_body_alias
    out3 = pl.pallas_call(
        body,
        grid=(t_half,),
        in_specs=in_specs,
        out_specs=pl.BlockSpec((1, EMB, batch),
                               lambda t, o=t_off: (t + o, 0, 0)),
        out_shape=jax.ShapeDtypeStruct((total_t, EMB, batch), jnp.float32),
        **kwargs,
    )(*args)
    return out3


def kernel(x, ks_table, pos_table, W1, b1, W2, b2):
    batch_dim, t_dim = x.shape
    xt = jnp.transpose(x).astype(jnp.int32)
    xt_perm = jnp.transpose(
        xt.reshape(t_dim, BT_PER_W, NW, 128), (0, 2, 1, 3))
    xt3 = xt_perm.reshape(t_dim, batch_dim // 128, 128)
    nsplit = 10
    th = t_dim // nsplit
    pos3 = pos_table.reshape(t_dim, 1, EMB)
    temps = [_sc_gather(ks_table, xt3[i * th:(i + 1) * th], batch_dim, th)
             for i in range(nsplit)]
    out3 = None
    for i in range(nsplit):
        out3 = _finish(temps[i], W1, b1, W2, b2, pos3[i * th:(i + 1) * th],
                       batch_dim, t_dim, i * th, alias=out3)
    out = jnp.transpose(out3, (2, 0, 1))
    return out


# speedup vs baseline: 1.0073x; 1.0073x over previous
import functools

import jax
import jax.numpy as jnp
from jax import lax
from jax.experimental import pallas as pl
from jax.experimental.pallas import tpu as pltpu
from jax.experimental.pallas import tpu_sc as plsc

EMB = 32
INNER = 128

NC = 2
NS = 16
NW = NC * NS
BT_PER_W = 4
BW = BT_PER_W * 128


def _sc_gather(table, xt3, batch, t_dim):
    mesh = plsc.VectorSubcoreMesh(core_axis_name="c", subcore_axis_name="s")

    @functools.partial(
        pl.kernel,
        mesh=mesh,
        out_type=jax.ShapeDtypeStruct((t_dim, batch * EMB // 128, 128),
                                      jnp.float32),
        scratch_types=[pltpu.VMEM((BT_PER_W, 128), jnp.int32) for _ in range(2)]
          + [pltpu.VMEM((BW, EMB), jnp.float32) for _ in range(2)]
          + [pltpu.SemaphoreType.DMA for _ in range(6)],
        compiler_params=pltpu.CompilerParams(use_tc_tiling_on_sc=False),
    )
    def body(tab_hbm, xt_hbm, out_hbm,
             idx0, idx1, rows0, rows1, si0, si1, sg0, sg1, ss0, ss1):
        idx = (idx0, idx1)
        rows = (rows0, rows1)
        sem_i = (si0, si1)
        sem_g = (sg0, sg1)
        sem_s = (ss0, ss1)
        wid = lax.axis_index("s") * NC + lax.axis_index("c")
        bt0 = wid * BT_PER_W
        r0 = wid * 128

        def fire_idx(t, p):
            pltpu.async_copy(xt_hbm.at[t, pl.ds(bt0, BT_PER_W)], idx[p],
                             sem_i[p])

        def wait_idx(p):
            pltpu.make_async_copy(xt_hbm.at[0, pl.ds(0, BT_PER_W)], idx[p],
                                  sem_i[p]).wait()

        def fire_gathers(p):
            for q in range(BT_PER_W):
                pltpu.async_copy(tab_hbm.at[idx[p].at[q]],
                                 rows[p].at[pl.ds(q * 128, 128)], sem_g[p])

        def wait_gathers(p):
            pltpu.make_async_copy(tab_hbm.at[pl.ds(0, BW)], rows[p],
                                  sem_g[p]).wait()

        def fire_store(t, p):
            for q in range(BT_PER_W):
                pltpu.async_copy(rows[p].at[pl.ds(q * 128, 128)],
                                 out_hbm.at[t, pl.ds(r0, 128),
                                            pl.ds(q * EMB, EMB)], sem_s[p])

        def wait_store(p):
            for q in range(BT_PER_W):
                pltpu.make_async_copy(rows[p].at[pl.ds(q * 128, 128)],
                                      out_hbm.at[0, pl.ds(0, 128),
                                                 pl.ds(0, EMB)],
                                      sem_s[p]).wait()

        def t_iter(t, p):
            q = 1 - p

            @pl.when(t + 1 < t_dim)
            def _():
                wait_idx(q)

                @pl.when(t >= 1)
                def _():
                    wait_store(q)

                fire_gathers(q)

            wait_gathers(p)

            @pl.when(t + 2 < t_dim)
            def _():
                fire_idx(t + 2, p)

            fire_store(t, p)

        pltpu.sync_copy(xt_hbm.at[0, pl.ds(bt0, BT_PER_W)], idx[0])
        fire_gathers(0)
        fire_idx(1, 1)

        def pair(m, carry):
            t_iter(2 * m, 0)
            t_iter(2 * m + 1, 1)
            return carry

        lax.fori_loop(0, t_dim // 2, pair, 0)
        wait_store(0)
        wait_store(1)

    return body(table, xt3)


def _finish_body(tmp_ref, w1_ref, w2_ref, pos_ref, b1_ref, b2_ref, out_ref):
    _finish_common(tmp_ref, w1_ref, w2_ref, pos_ref, b1_ref, b2_ref, out_ref)


def _finish_body_alias(tmp_ref, w1_ref, w2_ref, pos_ref, b1_ref, b2_ref,
                       alias_ref, out_ref):
    del alias_ref
    _finish_common(tmp_ref, w1_ref, w2_ref, pos_ref, b1_ref, b2_ref, out_ref)


def _finish_common(tmp_ref, w1_ref, w2_ref, pos_ref, b1_ref, b2_ref, out_ref):
    f32 = jnp.float32
    m = jnp.dot(w1_ref[...], w2_ref[...], preferred_element_type=f32)
    ptc = lax.dot_general(m, pos_ref[0], (((0,), (1,)), ((), ())),
                          preferred_element_type=f32)
    ccol = lax.dot_general(w2_ref[...], b1_ref[...], (((0,), (1,)), ((), ())),
                           preferred_element_type=f32)
    col = ptc + ccol + b2_ref[...]
    v = tmp_ref[0]
    qn = v.shape[0]
    for a in range(4):
        va = v[:, 32 * a:32 * (a + 1)]
        ga = lax.dot_general(m, va, (((0,), (1,)), ((), ())),
                             preferred_element_type=f32)
        out_ref[0, :, a * qn:(a + 1) * qn] = ga + col


def _finish(temp4, W1, b1, W2, b2, pos3, batch, total_t, t_off, alias=None):
    t_half = temp4.shape[0]
    in_specs = [
        pl.BlockSpec((1, batch * EMB // 128, 128), lambda t: (t, 0, 0)),
        pl.BlockSpec((EMB, INNER), lambda t: (0, 0)),
        pl.BlockSpec((INNER, EMB), lambda t: (0, 0)),
        pl.BlockSpec((1, 1, EMB), lambda t: (t, 0, 0)),
        pl.BlockSpec((1, INNER), lambda t: (0, 0)),
        pl.BlockSpec((EMB, 1), lambda t: (0, 0)),
    ]
    args = [temp4, W1, W2, pos3, b1.reshape(1, INNER), b2.reshape(EMB, 1)]
    kwargs = {}
    body = _finish_body
    if alias is not None:
        in_specs.append(pl.BlockSpec(memory_space=pl.ANY))
        args.append(alias)
        kwargs["input_output_aliases"] = {6: 0}
        body = _finish_body_alias
    out3 = pl.pallas_call(
        body,
        grid=(t_half,),
        in_specs=in_specs,
        out_specs=pl.BlockSpec((1, EMB, batch),
                               lambda t, o=t_off: (t + o, 0, 0)),
        out_shape=jax.ShapeDtypeStruct((total_t, EMB, batch), jnp.float32),
        **kwargs,
    )(*args)
    return out3


def kernel(x, ks_table, pos_table, W1, b1, W2, b2):
    batch_dim, t_dim = x.shape
    xt = jnp.transpose(x).astype(jnp.int32)
    xt_perm = jnp.transpose(
        xt.reshape(t_dim, BT_PER_W, NW, 128), (0, 2, 1, 3))
    xt3 = xt_perm.reshape(t_dim, batch_dim // 128, 128)
    nsplit = 5
    th = t_dim // nsplit
    pos3 = pos_table.reshape(t_dim, 1, EMB)
    temps = [_sc_gather(ks_table, xt3[i * th:(i + 1) * th], batch_dim, th)
             for i in range(nsplit)]
    out3 = None
    for i in range(nsplit):
        out3 = _finish(temps[i], W1, b1, W2, b2, pos3[i * th:(i + 1) * th],
                       batch_dim, t_dim, i * th, alias=out3)
    out = jnp.transpose(out3, (2, 0, 1))
    return out
